# Initial kernel scaffold; baseline (speedup 1.0000x reference)
import jax
import jax.numpy as jnp
from jax.experimental import pallas as pl


def kernel(query, reference_points, input_flatten, input_space_shape, input_level_start_idx, Wv, bv, Ws, bs, Wa, ba, Wo, bo):
    raise NotImplementedError("write your pallas kernel here")


# speedup vs baseline: 4.5050x; 4.5050x over previous
import functools

import jax
import jax.numpy as jnp
from jax import lax
from jax.experimental import pallas as pl
from jax.experimental.pallas import tpu as pltpu
from jax.experimental.pallas import tpu_sc as plsc

_M = 8
_L = 4
_P = 4
_D = 32
_LP4 = _L * _P * 4
_NW = 32


def _mmb(x, wt, b, br):
    r, k = x.shape
    nn = wt.shape[1]

    def body(xr, wr, brf, o):
        o[...] = jnp.dot(xr[...], wr[...],
                         preferred_element_type=jnp.float32) + brf[...]

    return pl.pallas_call(
        body,
        grid=(r // br,),
        in_specs=[
            pl.BlockSpec((br, k), lambda i: (i, 0)),
            pl.BlockSpec((k, nn), lambda i: (0, 0)),
            pl.BlockSpec((1, nn), lambda i: (0, 0)),
        ],
        out_specs=pl.BlockSpec((br, nn), lambda i: (i, 0)),
        out_shape=jax.ShapeDtypeStruct((r, nn), jnp.float32),
    )(x, wt, b)


def _sample_body(q, wsx, bsx, wsy, bsy, wa, bav, g, rfx, rfy,
                 wf, hf, wi, hi, st, mv, nb,
                 i00, i10, i01, i11, w00, w10, w01, w11):
    qq = q[...]
    ox = jnp.dot(qq, wsx[...], preferred_element_type=jnp.float32) + bsx[...]
    oy = jnp.dot(qq, wsy[...], preferred_element_type=jnp.float32) + bsy[...]
    a = jnp.dot(qq, wa[...], preferred_element_type=jnp.float32) + bav[...]
    e = jnp.exp(a - jnp.max(a, axis=-1, keepdims=True))
    s = jnp.dot(e, g[...], preferred_element_type=jnp.float32)
    at = e / s

    x = rfx[...] * wf[...] + ox - 0.5
    y = rfy[...] * hf[...] + oy - 0.5
    x0f = jnp.floor(x)
    y0f = jnp.floor(y)
    fx = x - x0f
    fy = y - y0f
    x0 = x0f.astype(jnp.int32)
    y0 = y0f.astype(jnp.int32)
    x1 = x0 + 1
    y1 = y0 + 1
    wim1 = wi[...] - 1
    him1 = hi[...] - 1
    vx0 = (x0 >= 0) & (x0 <= wim1)
    vx1 = (x1 >= 0) & (x1 <= wim1)
    vy0 = (y0 >= 0) & (y0 <= him1)
    vy1 = (y1 >= 0) & (y1 <= him1)
    cx0 = jnp.clip(x0, 0, wim1)
    cx1 = jnp.clip(x1, 0, wim1)
    cy0 = jnp.clip(y0, 0, him1)
    cy1 = jnp.clip(y1, 0, him1)

    base = nb[...]
    stt = st[...]
    mvv = mv[...]
    wiv = wi[...]

    def mkidx(cx, cy, v):
        pos = stt + cy * wiv + cx
        ridx = (base + pos) * _M + mvv
        return jnp.where(v, ridx, 0)

    v00 = vx0 & vy0
    v10 = vx1 & vy0
    v01 = vx0 & vy1
    v11 = vx1 & vy1
    i00[...] = mkidx(cx0, cy0, v00)
    i10[...] = mkidx(cx1, cy0, v10)
    i01[...] = mkidx(cx0, cy1, v01)
    i11[...] = mkidx(cx1, cy1, v11)
    gx0 = 1.0 - fx
    gy0 = 1.0 - fy
    w00[...] = jnp.where(v00, at * gx0 * gy0, 0.0)
    w10[...] = jnp.where(v10, at * fx * gy0, 0.0)
    w01[...] = jnp.where(v01, at * gx0 * fy, 0.0)
    w11[...] = jnp.where(v11, at * fx * fy, 0.0)


def _sample(q2, wsxt, bsx, wsyt, bsy, wat, ba2, g, rfx, rfy,
            wf, hf, wi, hi, st, mv, nb):
    r = q2.shape[0]
    br = r // 2
    c = q2.shape[1]
    s128 = 128

    def rowspec(w):
        return pl.BlockSpec((br, w), lambda i: (i, 0))

    def full(a, b):
        return pl.BlockSpec((a, b), lambda i: (0, 0))

    io = jax.ShapeDtypeStruct((r, s128), jnp.int32)
    wo = jax.ShapeDtypeStruct((r, s128), jnp.float32)
    return pl.pallas_call(
        _sample_body,
        grid=(2,),
        in_specs=[
            rowspec(c), full(c, s128), full(1, s128), full(c, s128),
            full(1, s128), full(c, s128), full(1, s128), full(s128, s128),
            rowspec(s128), rowspec(s128), full(1, s128), full(1, s128),
            full(1, s128), full(1, s128), full(1, s128), full(1, s128),
            rowspec(s128),
        ],
        out_specs=[rowspec(s128)] * 8,
        out_shape=[io, io, io, io, wo, wo, wo, wo],
    )(q2, wsxt, bsx, wsyt, bsy, wat, ba2, g, rfx, rfy,
      wf, hf, wi, hi, st, mv, nb)


def _sc_gather(table, idx2d, wgt, rows):
    per = rows // _NW
    ch = 16
    nchunk = per // ch
    rpc = ch * _LP4
    slabs = rpc // 128

    mesh = plsc.VectorSubcoreMesh(core_axis_name="c", subcore_axis_name="s")

    @functools.partial(
        pl.kernel,
        out_type=jax.ShapeDtypeStruct((rows, _D), jnp.float32),
        mesh=mesh,
        compiler_params=pltpu.CompilerParams(use_tc_tiling_on_sc=False),
        scratch_types=[
            pltpu.VMEM((slabs, 128), jnp.int32),
            pltpu.VMEM((rpc, _D), jnp.float32),
            pltpu.VMEM((rpc,), jnp.float32),
            pltpu.VMEM((ch, _D), jnp.float32),
            pltpu.SemaphoreType.DMA,
        ],
    )
    def k(tab, idxh, wh, outh, idx_v, rows_v, w_v, out_v, sem):
        wid = lax.axis_index("s") * 2 + lax.axis_index("c")
        base0 = wid * per

        def chunk(t, carry):
            base = pl.multiple_of(base0 + t * ch, ch)
            slab0 = pl.multiple_of(base * _LP4 // 128, slabs)
            pltpu.sync_copy(idxh.at[pl.ds(slab0, slabs)], idx_v)
            pltpu.sync_copy(wh.at[pl.ds(pl.multiple_of(base * _LP4, rpc), rpc)],
                            w_v)
            cps = [
                pltpu.async_copy(tab.at[idx_v.at[sl]],
                                 rows_v.at[pl.ds(sl * 128, 128)], sem)
                for sl in range(slabs)
            ]
            for cp in cps:
                cp.wait()

            def rbody(rr, cr):
                k0 = rr * _LP4
                a0 = jnp.zeros((16,), jnp.float32)
                a1 = jnp.zeros((16,), jnp.float32)
                for jg in range(_LP4 // 16):
                    wvec = w_v[pl.ds(k0 + jg * 16, 16)]
                    for jj in range(16):
                        kk = k0 + jg * 16 + jj
                        wsc = wvec[jj]
                        a0 = a0 + wsc * rows_v[kk, 0:16]
                        a1 = a1 + wsc * rows_v[kk, 16:32]
                out_v[rr, 0:16] = a0
                out_v[rr, 16:32] = a1
                return cr

            lax.fori_loop(0, ch, rbody, 0)
            pltpu.sync_copy(out_v, outh.at[pl.ds(pl.multiple_of(base, ch), ch)])
            return carry

        lax.fori_loop(0, nchunk, chunk, 0)

    return k(table, idx2d, wgt)


def kernel(query, reference_points, input_flatten, input_space_shape,
           input_level_start_idx, Wv, bv, Ws, bs, Wa, ba, Wo, bo):
    n, lq, c = query.shape
    len_in = input_flatten.shape[1]
    r = n * lq
    rows = r * _M

    q2 = query.reshape(r, c)
    x2 = input_flatten.reshape(n * len_in, c)

    val = _mmb(x2, Wv.T, bv.reshape(1, c), br=512)
    table = val.reshape(n * len_in * _M, _D)

    wst = Ws.T
    wsxt = wst[:, 0::2]
    wsyt = wst[:, 1::2]
    bsx = bs[0::2].reshape(1, 128)
    bsy = bs[1::2].reshape(1, 128)
    wat = Wa.T
    ba2 = ba.reshape(1, 128)
    lane = jnp.arange(128, dtype=jnp.int32)
    g = (lane[:, None] // 16 == lane[None, :] // 16).astype(jnp.float32)

    rp = reference_points.reshape(r, _L, 2)
    rfx = jnp.tile(jnp.repeat(rp[:, :, 0], _P, axis=1), (1, _M))
    rfy = jnp.tile(jnp.repeat(rp[:, :, 1], _P, axis=1), (1, _M))

    def t128(v):
        return jnp.tile(jnp.repeat(v, _P), (_M,)).reshape(1, 128)

    wf = t128(input_space_shape[:, 1].astype(jnp.float32))
    hf = t128(input_space_shape[:, 0].astype(jnp.float32))
    wi = t128(input_space_shape[:, 1])
    hi = t128(input_space_shape[:, 0])
    st = t128(input_level_start_idx)
    mv = jnp.repeat(jnp.arange(_M, dtype=jnp.int32), _L * _P).reshape(1, 128)
    nb = jnp.broadcast_to(
        ((jnp.arange(r, dtype=jnp.int32) // lq) * len_in)[:, None], (r, 128))

    i00, i10, i01, i11, w00, w10, w01, w11 = _sample(
        q2, wsxt, bsx, wsyt, bsy, wat, ba2, g, rfx, rfy,
        wf, hf, wi, hi, st, mv, nb)

    idx = jnp.stack([i00, i10, i01, i11], axis=-1).reshape(rows, _LP4)
    wgt = jnp.stack([w00, w10, w01, w11], axis=-1).reshape(rows, _LP4)
    rows_pad = -(-rows // (_NW * 16)) * (_NW * 16)
    pad = rows_pad - rows
    idx = jnp.concatenate([idx, jnp.zeros((pad, _LP4), jnp.int32)], axis=0)
    wgt = jnp.concatenate([wgt, jnp.zeros((pad, _LP4), jnp.float32)], axis=0)
    idx2d = idx.reshape(rows_pad * _LP4 // 128, 128)

    out_sc = _sc_gather(table, idx2d, wgt.reshape(-1), rows_pad)[:rows]

    out = _mmb(out_sc.reshape(r, c), Wo.T, bo.reshape(1, c), br=720)
    return out.reshape(n, lq, c)


# speedup vs baseline: 6.2886x; 1.3959x over previous
import functools

import jax
import jax.numpy as jnp
from jax import lax
from jax.experimental import pallas as pl
from jax.experimental.pallas import tpu as pltpu
from jax.experimental.pallas import tpu_sc as plsc

_M = 8
_L = 4
_P = 4
_D = 32
_LP4 = _L * _P * 4
_NW = 32


def _mmb(x, wt, b, br, out_dtype=jnp.float32):
    r, k = x.shape
    nn = wt.shape[1]

    def body(xr, wr, brf, o):
        o[...] = (jnp.dot(xr[...], wr[...],
                          preferred_element_type=jnp.float32)
                  + brf[...]).astype(out_dtype)

    return pl.pallas_call(
        body,
        grid=(r // br,),
        in_specs=[
            pl.BlockSpec((br, k), lambda i: (i, 0)),
            pl.BlockSpec((k, nn), lambda i: (0, 0)),
            pl.BlockSpec((1, nn), lambda i: (0, 0)),
        ],
        out_specs=pl.BlockSpec((br, nn), lambda i: (i, 0)),
        out_shape=jax.ShapeDtypeStruct((r, nn), out_dtype),
    )(x, wt, b)


def _sample_body(q, wsx, bsx, wsy, bsy, wa, bav, g, rfx, rfy,
                 wf, hf, wi, hi, st, mv, nb,
                 i00, i10, i01, i11, w00, w10, w01, w11):
    qq = q[...]
    ox = jnp.dot(qq, wsx[...], preferred_element_type=jnp.float32) + bsx[...]
    oy = jnp.dot(qq, wsy[...], preferred_element_type=jnp.float32) + bsy[...]
    a = jnp.dot(qq, wa[...], preferred_element_type=jnp.float32) + bav[...]
    e = jnp.exp(a - jnp.max(a, axis=-1, keepdims=True))
    s = jnp.dot(e, g[...], preferred_element_type=jnp.float32)
    at = e / s

    x = rfx[...] * wf[...] + ox - 0.5
    y = rfy[...] * hf[...] + oy - 0.5
    x0f = jnp.floor(x)
    y0f = jnp.floor(y)
    fx = x - x0f
    fy = y - y0f
    x0 = x0f.astype(jnp.int32)
    y0 = y0f.astype(jnp.int32)
    x1 = x0 + 1
    y1 = y0 + 1
    wim1 = wi[...] - 1
    him1 = hi[...] - 1
    vx0 = (x0 >= 0) & (x0 <= wim1)
    vx1 = (x1 >= 0) & (x1 <= wim1)
    vy0 = (y0 >= 0) & (y0 <= him1)
    vy1 = (y1 >= 0) & (y1 <= him1)
    cx0 = jnp.clip(x0, 0, wim1)
    cx1 = jnp.clip(x1, 0, wim1)
    cy0 = jnp.clip(y0, 0, him1)
    cy1 = jnp.clip(y1, 0, him1)

    base = nb[...]
    stt = st[...]
    mvv = mv[...]
    wiv = wi[...]

    def mkidx(cx, cy, v):
        pos = stt + cy * wiv + cx
        ridx = (base + pos) * _M + mvv
        return jnp.where(v, ridx, 0)

    v00 = vx0 & vy0
    v10 = vx1 & vy0
    v01 = vx0 & vy1
    v11 = vx1 & vy1
    i00[...] = mkidx(cx0, cy0, v00)
    i10[...] = mkidx(cx1, cy0, v10)
    i01[...] = mkidx(cx0, cy1, v01)
    i11[...] = mkidx(cx1, cy1, v11)
    gx0 = 1.0 - fx
    gy0 = 1.0 - fy
    w00[...] = jnp.where(v00, at * gx0 * gy0, 0.0)
    w10[...] = jnp.where(v10, at * fx * gy0, 0.0)
    w01[...] = jnp.where(v01, at * gx0 * fy, 0.0)
    w11[...] = jnp.where(v11, at * fx * fy, 0.0)


def _sample(q2, wsxt, bsx, wsyt, bsy, wat, ba2, g, rfx, rfy,
            wf, hf, wi, hi, st, mv, nb):
    r = q2.shape[0]
    br = r // 2
    c = q2.shape[1]
    s128 = 128

    def rowspec(w):
        return pl.BlockSpec((br, w), lambda i: (i, 0))

    def full(a, b):
        return pl.BlockSpec((a, b), lambda i: (0, 0))

    io = jax.ShapeDtypeStruct((r, s128), jnp.int32)
    wo = jax.ShapeDtypeStruct((r, s128), jnp.float32)
    return pl.pallas_call(
        _sample_body,
        grid=(2,),
        in_specs=[
            rowspec(c), full(c, s128), full(1, s128), full(c, s128),
            full(1, s128), full(c, s128), full(1, s128), full(s128, s128),
            rowspec(s128), rowspec(s128), full(1, s128), full(1, s128),
            full(1, s128), full(1, s128), full(1, s128), full(1, s128),
            rowspec(s128),
        ],
        out_specs=[rowspec(s128)] * 8,
        out_shape=[io, io, io, io, wo, wo, wo, wo],
    )(q2, wsxt, bsx, wsyt, bsy, wat, ba2, g, rfx, rfy,
      wf, hf, wi, hi, st, mv, nb)


def _sc_gather(table, idx2d, wgt, rows):
    per = rows // _NW
    ch = 16
    nit = per // (2 * ch)
    rpc = ch * _LP4
    slabs = rpc // 128

    mesh = plsc.VectorSubcoreMesh(core_axis_name="c", subcore_axis_name="s")

    @functools.partial(
        pl.kernel,
        out_type=jax.ShapeDtypeStruct((rows, _D), jnp.float32),
        mesh=mesh,
        compiler_params=pltpu.CompilerParams(use_tc_tiling_on_sc=False,
                                             needs_layout_passes=False),
        scratch_types=[
            pltpu.VMEM((2, slabs, 128), jnp.int32),
            pltpu.VMEM((2, rpc, _D), jnp.bfloat16),
            pltpu.VMEM((2, rpc), jnp.float32),
            pltpu.VMEM((ch, _D), jnp.float32),
            pltpu.SemaphoreType.DMA,
            pltpu.SemaphoreType.DMA,
        ],
    )
    def k(tab, idxh, wh, outh, idx_v, rows_v, w_v, out_v, sem0, sem1):
        wid = lax.axis_index("s") * 2 + lax.axis_index("c")
        base0 = wid * per
        sems = (sem0, sem1)

        def fetch_fire(t, b):
            base = pl.multiple_of(base0 + t * ch, ch)
            slab0 = pl.multiple_of(base * _LP4 // 128, slabs)
            pltpu.sync_copy(idxh.at[pl.ds(slab0, slabs)], idx_v.at[b])
            pltpu.sync_copy(wh.at[pl.ds(pl.multiple_of(base * _LP4, rpc), rpc)],
                            w_v.at[b])
            for sl in range(slabs):
                pltpu.async_copy(tab.at[idx_v.at[b].at[sl]],
                                 rows_v.at[b].at[pl.ds(sl * 128, 128)],
                                 sems[b])

        def drain(b):
            pltpu.make_async_copy(tab.at[pl.ds(0, rpc)], rows_v.at[b],
                                  sems[b]).wait()

        def compute_store(t, b):
            rv = rows_v.at[b]
            wv = w_v.at[b]

            def rbody(rr, cr):
                k0 = rr * _LP4
                a0 = jnp.zeros((16,), jnp.float32)
                a1 = jnp.zeros((16,), jnp.float32)
                for jg in range(_LP4 // 16):
                    wvec = wv[pl.ds(k0 + jg * 16, 16)]
                    for jj in range(16):
                        kk = k0 + jg * 16 + jj
                        ev, od = plsc.unpack(
                            rv[kk, :], format=plsc.PackFormat.INTERLEAVED)
                        wsc = wvec[jj]
                        a0 = a0 + wsc * ev
                        a1 = a1 + wsc * od
                out_v[rr, 0:16] = a0
                out_v[rr, 16:32] = a1
                return cr

            lax.fori_loop(0, ch, rbody, 0)
            base = pl.multiple_of(base0 + t * ch, ch)
            pltpu.sync_copy(out_v, outh.at[pl.ds(base, ch)])

        fetch_fire(0, 0)

        def it(i, carry):
            t0 = i * 2
            fetch_fire(t0 + 1, 1)
            drain(0)
            compute_store(t0, 0)

            @pl.when(i < nit - 1)
            def _():
                fetch_fire(t0 + 2, 0)

            drain(1)
            compute_store(t0 + 1, 1)
            return carry

        lax.fori_loop(0, nit, it, 0)

    return k(table, idx2d, wgt)


def kernel(query, reference_points, input_flatten, input_space_shape,
           input_level_start_idx, Wv, bv, Ws, bs, Wa, ba, Wo, bo):
    n, lq, c = query.shape
    len_in = input_flatten.shape[1]
    r = n * lq
    rows = r * _M

    q2 = query.reshape(r, c)
    x2 = input_flatten.reshape(n * len_in, c)

    val = _mmb(x2, Wv.T, bv.reshape(1, c), br=512, out_dtype=jnp.bfloat16)
    table = val.reshape(n * len_in * _M, _D)

    wst = Ws.T
    wsxt = wst[:, 0::2]
    wsyt = wst[:, 1::2]
    bsx = bs[0::2].reshape(1, 128)
    bsy = bs[1::2].reshape(1, 128)
    wat = Wa.T
    ba2 = ba.reshape(1, 128)
    lane = jnp.arange(128, dtype=jnp.int32)
    g = (lane[:, None] // 16 == lane[None, :] // 16).astype(jnp.float32)

    rp = reference_points.reshape(r, _L, 2)
    rfx = jnp.tile(jnp.repeat(rp[:, :, 0], _P, axis=1), (1, _M))
    rfy = jnp.tile(jnp.repeat(rp[:, :, 1], _P, axis=1), (1, _M))

    def t128(v):
        return jnp.tile(jnp.repeat(v, _P), (_M,)).reshape(1, 128)

    wf = t128(input_space_shape[:, 1].astype(jnp.float32))
    hf = t128(input_space_shape[:, 0].astype(jnp.float32))
    wi = t128(input_space_shape[:, 1])
    hi = t128(input_space_shape[:, 0])
    st = t128(input_level_start_idx)
    mv = jnp.repeat(jnp.arange(_M, dtype=jnp.int32), _L * _P).reshape(1, 128)
    nb = jnp.broadcast_to(
        ((jnp.arange(r, dtype=jnp.int32) // lq) * len_in)[:, None], (r, 128))

    i00, i10, i01, i11, w00, w10, w01, w11 = _sample(
        q2, wsxt, bsx, wsyt, bsy, wat, ba2, g, rfx, rfy,
        wf, hf, wi, hi, st, mv, nb)

    idx = jnp.stack([i00, i10, i01, i11], axis=-1).reshape(rows, _LP4)
    wgt = jnp.stack([w00, w10, w01, w11], axis=-1).reshape(rows, _LP4)
    rows_pad = -(-rows // (_NW * 32)) * (_NW * 32)
    pad = rows_pad - rows
    idx = jnp.concatenate([idx, jnp.zeros((pad, _LP4), jnp.int32)], axis=0)
    wgt = jnp.concatenate([wgt, jnp.zeros((pad, _LP4), jnp.float32)], axis=0)
    idx2d = idx.reshape(rows_pad * _LP4 // 128, 128)

    out_sc = _sc_gather(table, idx2d, wgt.reshape(-1), rows_pad)[:rows]

    half = _D // 2
    pcol = jnp.arange(_D)
    chan = jnp.where(pcol < half, 2 * pcol, 2 * (pcol - half) + 1)
    perm = (jnp.arange(c) // _D) * _D + chan[jnp.arange(c) % _D]
    wot_perm = Wo.T[perm, :]
    out = _mmb(out_sc.reshape(r, c), wot_perm, bo.reshape(1, c), br=720)
    return out.reshape(n, lq, c)


# speedup vs baseline: 9.4450x; 1.5019x over previous
import functools

import jax
import jax.numpy as jnp
from jax import lax
from jax.experimental import pallas as pl
from jax.experimental.pallas import tpu as pltpu
from jax.experimental.pallas import tpu_sc as plsc

_M = 8
_L = 4
_P = 4
_D = 32
_LP4 = _L * _P * 4
_NW = 32


def _mmb(x, wt, b, br, out_dtype=jnp.float32):
    r, k = x.shape
    nn = wt.shape[1]

    def body(xr, wr, brf, o):
        o[...] = (jnp.dot(xr[...], wr[...],
                          preferred_element_type=jnp.float32)
                  + brf[...]).astype(out_dtype)

    return pl.pallas_call(
        body,
        grid=(r // br,),
        in_specs=[
            pl.BlockSpec((br, k), lambda i: (i, 0)),
            pl.BlockSpec((k, nn), lambda i: (0, 0)),
            pl.BlockSpec((1, nn), lambda i: (0, 0)),
        ],
        out_specs=pl.BlockSpec((br, nn), lambda i: (i, 0)),
        out_shape=jax.ShapeDtypeStruct((r, nn), out_dtype),
    )(x, wt, b)


def _sample_body(q, wsx, bsx, wsy, bsy, wa, bav, g, rfx, rfy,
                 wf, hf, wi, hi, st, mv, nb,
                 i00, i10, i01, i11, w00, w10, w01, w11):
    qq = q[...]
    ox = jnp.dot(qq, wsx[...], preferred_element_type=jnp.float32) + bsx[...]
    oy = jnp.dot(qq, wsy[...], preferred_element_type=jnp.float32) + bsy[...]
    a = jnp.dot(qq, wa[...], preferred_element_type=jnp.float32) + bav[...]
    e = jnp.exp(a - jnp.max(a, axis=-1, keepdims=True))
    s = jnp.dot(e, g[...], preferred_element_type=jnp.float32)
    at = e / s

    x = rfx[...] * wf[...] + ox - 0.5
    y = rfy[...] * hf[...] + oy - 0.5
    x0f = jnp.floor(x)
    y0f = jnp.floor(y)
    fx = x - x0f
    fy = y - y0f
    x0 = x0f.astype(jnp.int32)
    y0 = y0f.astype(jnp.int32)
    x1 = x0 + 1
    y1 = y0 + 1
    wim1 = wi[...] - 1
    him1 = hi[...] - 1
    vx0 = (x0 >= 0) & (x0 <= wim1)
    vx1 = (x1 >= 0) & (x1 <= wim1)
    vy0 = (y0 >= 0) & (y0 <= him1)
    vy1 = (y1 >= 0) & (y1 <= him1)
    cx0 = jnp.clip(x0, 0, wim1)
    cx1 = jnp.clip(x1, 0, wim1)
    cy0 = jnp.clip(y0, 0, him1)
    cy1 = jnp.clip(y1, 0, him1)

    base = nb[...]
    stt = st[...]
    mvv = mv[...]
    wiv = wi[...]

    def mkidx(cx, cy, v):
        pos = stt + cy * wiv + cx
        ridx = (base + pos) * _M + mvv
        return jnp.where(v, ridx, 0)

    v00 = vx0 & vy0
    v10 = vx1 & vy0
    v01 = vx0 & vy1
    v11 = vx1 & vy1
    i00[...] = mkidx(cx0, cy0, v00)
    i10[...] = mkidx(cx1, cy0, v10)
    i01[...] = mkidx(cx0, cy1, v01)
    i11[...] = mkidx(cx1, cy1, v11)
    gx0 = 1.0 - fx
    gy0 = 1.0 - fy
    w00[...] = jnp.where(v00, at * gx0 * gy0, 0.0)
    w10[...] = jnp.where(v10, at * fx * gy0, 0.0)
    w01[...] = jnp.where(v01, at * gx0 * fy, 0.0)
    w11[...] = jnp.where(v11, at * fx * fy, 0.0)


def _sample(q2, wsxt, bsx, wsyt, bsy, wat, ba2, g, rfx, rfy,
            wf, hf, wi, hi, st, mv, nb):
    r = q2.shape[0]
    br = r // 2
    c = q2.shape[1]
    s128 = 128

    def rowspec(w):
        return pl.BlockSpec((br, w), lambda i: (i, 0))

    def full(a, b):
        return pl.BlockSpec((a, b), lambda i: (0, 0))

    io = jax.ShapeDtypeStruct((r, s128), jnp.int32)
    wo = jax.ShapeDtypeStruct((r, s128), jnp.float32)
    return pl.pallas_call(
        _sample_body,
        grid=(2,),
        in_specs=[
            rowspec(c), full(c, s128), full(1, s128), full(c, s128),
            full(1, s128), full(c, s128), full(1, s128), full(s128, s128),
            rowspec(s128), rowspec(s128), full(1, s128), full(1, s128),
            full(1, s128), full(1, s128), full(1, s128), full(1, s128),
            rowspec(s128),
        ],
        out_specs=[rowspec(s128)] * 8,
        out_shape=[io, io, io, io, wo, wo, wo, wo],
    )(q2, wsxt, bsx, wsyt, bsy, wat, ba2, g, rfx, rfy,
      wf, hf, wi, hi, st, mv, nb)


def _sc_gather(table, idxs, wgts, rq):
    ch = 2
    perq = rq // _NW
    nit = perq // (2 * ch)
    rpc = ch * 128 * 4
    rows = rq * _M

    mesh = plsc.VectorSubcoreMesh(core_axis_name="c", subcore_axis_name="s")

    @functools.partial(
        pl.kernel,
        out_type=jax.ShapeDtypeStruct((rows, _D), jnp.float32),
        mesh=mesh,
        compiler_params=pltpu.CompilerParams(use_tc_tiling_on_sc=False,
                                             needs_layout_passes=False),
        scratch_types=[
            pltpu.VMEM((2, 8, 128), jnp.int32),
            pltpu.VMEM((2, rpc, _D), jnp.bfloat16),
            pltpu.VMEM((2, 8, 128), jnp.float32),
            pltpu.VMEM((ch * _M, _D), jnp.float32),
            pltpu.SemaphoreType.DMA,
            pltpu.SemaphoreType.DMA,
        ],
    )
    def k(tab, i0h, i1h, i2h, i3h, w0h, w1h, w2h, w3h, outh,
          idx_v, rows_v, w_v, out_v, sem0, sem1):
        wid = lax.axis_index("s") * 2 + lax.axis_index("c")
        qbase = wid * perq
        sems = (sem0, sem1)
        ihs = (i0h, i1h, i2h, i3h)
        whs = (w0h, w1h, w2h, w3h)

        def fetch_fire(t, b):
            q0 = pl.multiple_of(qbase + t * ch, ch)
            for c in range(4):
                pltpu.sync_copy(ihs[c].at[pl.ds(q0, ch)],
                                idx_v.at[b].at[pl.ds(c * ch, ch)])
                pltpu.sync_copy(whs[c].at[pl.ds(q0, ch)],
                                w_v.at[b].at[pl.ds(c * ch, ch)])
            for sl in range(4 * ch):
                pltpu.async_copy(tab.at[idx_v.at[b].at[sl]],
                                 rows_v.at[b].at[pl.ds(sl * 128, 128)],
                                 sems[b])

        def drain(b):
            pltpu.make_async_copy(tab.at[pl.ds(0, rpc)], rows_v.at[b],
                                  sems[b]).wait()

        def compute_store(t, b):
            rv = rows_v.at[b]
            wv = w_v.at[b]

            def rbody(rr, cr):
                qq = rr // _M
                m16 = (rr - qq * _M) * 16
                a0 = jnp.zeros((16,), jnp.float32)
                a1 = jnp.zeros((16,), jnp.float32)
                for c in range(4):
                    s = c * ch + qq
                    wvec = wv[s, pl.ds(m16, 16)]
                    k0 = s * 128 + m16
                    for jj in range(16):
                        ev, od = plsc.unpack(
                            rv[k0 + jj, :],
                            format=plsc.PackFormat.INTERLEAVED)
                        wsc = wvec[jj]
                        a0 = a0 + wsc * ev
                        a1 = a1 + wsc * od
                out_v[rr, 0:16] = a0
                out_v[rr, 16:32] = a1
                return cr

            lax.fori_loop(0, ch * _M, rbody, 0)
            q0 = pl.multiple_of(qbase + t * ch, ch)
            pltpu.sync_copy(out_v, outh.at[pl.ds(q0 * _M, ch * _M)])

        fetch_fire(0, 0)

        def it(i, carry):
            t0 = i * 2
            fetch_fire(t0 + 1, 1)
            drain(0)
            compute_store(t0, 0)

            @pl.when(i < nit - 1)
            def _():
                fetch_fire(t0 + 2, 0)

            drain(1)
            compute_store(t0 + 1, 1)
            return carry

        lax.fori_loop(0, nit, it, 0)

    return k(table, *idxs, *wgts)


def kernel(query, reference_points, input_flatten, input_space_shape,
           input_level_start_idx, Wv, bv, Ws, bs, Wa, ba, Wo, bo):
    n, lq, c = query.shape
    len_in = input_flatten.shape[1]
    r = n * lq
    rows = r * _M

    q2 = query.reshape(r, c)
    x2 = input_flatten.reshape(n * len_in, c)

    val = _mmb(x2, Wv.T, bv.reshape(1, c), br=512, out_dtype=jnp.bfloat16)
    table = val.reshape(n * len_in * _M, _D)

    wst = Ws.T
    wsxt = wst[:, 0::2]
    wsyt = wst[:, 1::2]
    bsx = bs[0::2].reshape(1, 128)
    bsy = bs[1::2].reshape(1, 128)
    wat = Wa.T
    ba2 = ba.reshape(1, 128)
    lane = jnp.arange(128, dtype=jnp.int32)
    g = (lane[:, None] // 16 == lane[None, :] // 16).astype(jnp.float32)

    rp = reference_points.reshape(r, _L, 2)
    rfx = jnp.tile(jnp.repeat(rp[:, :, 0], _P, axis=1), (1, _M))
    rfy = jnp.tile(jnp.repeat(rp[:, :, 1], _P, axis=1), (1, _M))

    def t128(v):
        return jnp.tile(jnp.repeat(v, _P), (_M,)).reshape(1, 128)

    wf = t128(input_space_shape[:, 1].astype(jnp.float32))
    hf = t128(input_space_shape[:, 0].astype(jnp.float32))
    wi = t128(input_space_shape[:, 1])
    hi = t128(input_space_shape[:, 0])
    st = t128(input_level_start_idx)
    mv = jnp.repeat(jnp.arange(_M, dtype=jnp.int32), _L * _P).reshape(1, 128)
    nb = jnp.broadcast_to(
        ((jnp.arange(r, dtype=jnp.int32) // lq) * len_in)[:, None], (r, 128))

    i00, i10, i01, i11, w00, w10, w01, w11 = _sample(
        q2, wsxt, bsx, wsyt, bsy, wat, ba2, g, rfx, rfy,
        wf, hf, wi, hi, st, mv, nb)

    rq = -(-r // (_NW * 4)) * (_NW * 4)
    padq = rq - r

    def padz(a, dt):
        return jnp.concatenate([a, jnp.zeros((padq, 128), dt)], axis=0)

    idxs = [padz(a, jnp.int32) for a in (i00, i10, i01, i11)]
    wgts = [padz(a, jnp.float32) for a in (w00, w10, w01, w11)]

    out_sc = _sc_gather(table, idxs, wgts, rq)[:rows]

    half = _D // 2
    pcol = jnp.arange(_D)
    chan = jnp.where(pcol < half, 2 * pcol, 2 * (pcol - half) + 1)
    perm = (jnp.arange(c) // _D) * _D + chan[jnp.arange(c) % _D]
    wot_perm = Wo.T[perm, :]
    out = _mmb(out_sc.reshape(r, c), wot_perm, bo.reshape(1, c), br=720)
    return out.reshape(n, lq, c)


# speedup vs baseline: 9.5128x; 1.0072x over previous
import functools

import jax
import jax.numpy as jnp
from jax import lax
from jax.experimental import pallas as pl
from jax.experimental.pallas import tpu as pltpu
from jax.experimental.pallas import tpu_sc as plsc

_M = 8
_L = 4
_P = 4
_D = 32
_LP4 = _L * _P * 4
_NW = 32


def _mmb(x, wt, b, br, out_dtype=jnp.float32):
    r, k = x.shape
    nn = wt.shape[1]

    def body(xr, wr, brf, o):
        o[...] = (jnp.dot(xr[...], wr[...],
                          preferred_element_type=jnp.float32)
                  + brf[...]).astype(out_dtype)

    return pl.pallas_call(
        body,
        grid=(r // br,),
        in_specs=[
            pl.BlockSpec((br, k), lambda i: (i, 0)),
            pl.BlockSpec((k, nn), lambda i: (0, 0)),
            pl.BlockSpec((1, nn), lambda i: (0, 0)),
        ],
        out_specs=pl.BlockSpec((br, nn), lambda i: (i, 0)),
        out_shape=jax.ShapeDtypeStruct((r, nn), out_dtype),
    )(x, wt, b)


def _sample_body(q, wsx, bsx, wsy, bsy, wa, bav, g, rfx, rfy,
                 wf, hf, wi, hi, st, mv, nb,
                 i00, i10, i01, i11, w00, w10, w01, w11):
    qq = q[...]
    ox = jnp.dot(qq, wsx[...], preferred_element_type=jnp.float32) + bsx[...]
    oy = jnp.dot(qq, wsy[...], preferred_element_type=jnp.float32) + bsy[...]
    a = jnp.dot(qq, wa[...], preferred_element_type=jnp.float32) + bav[...]
    e = jnp.exp(a - jnp.max(a, axis=-1, keepdims=True))
    s = jnp.dot(e, g[...], preferred_element_type=jnp.float32)
    at = e / s

    x = rfx[...] * wf[...] + ox - 0.5
    y = rfy[...] * hf[...] + oy - 0.5
    x0f = jnp.floor(x)
    y0f = jnp.floor(y)
    fx = x - x0f
    fy = y - y0f
    x0 = x0f.astype(jnp.int32)
    y0 = y0f.astype(jnp.int32)
    x1 = x0 + 1
    y1 = y0 + 1
    wim1 = wi[...] - 1
    him1 = hi[...] - 1
    vx0 = (x0 >= 0) & (x0 <= wim1)
    vx1 = (x1 >= 0) & (x1 <= wim1)
    vy0 = (y0 >= 0) & (y0 <= him1)
    vy1 = (y1 >= 0) & (y1 <= him1)
    cx0 = jnp.clip(x0, 0, wim1)
    cx1 = jnp.clip(x1, 0, wim1)
    cy0 = jnp.clip(y0, 0, him1)
    cy1 = jnp.clip(y1, 0, him1)

    base = nb[...]
    stt = st[...]
    mvv = mv[...]
    wiv = wi[...]

    def mkidx(cx, cy, v):
        pos = stt + cy * wiv + cx
        ridx = (base + pos) * _M + mvv
        return jnp.where(v, ridx, 0)

    v00 = vx0 & vy0
    v10 = vx1 & vy0
    v01 = vx0 & vy1
    v11 = vx1 & vy1
    i00[...] = mkidx(cx0, cy0, v00)
    i10[...] = mkidx(cx1, cy0, v10)
    i01[...] = mkidx(cx0, cy1, v01)
    i11[...] = mkidx(cx1, cy1, v11)
    gx0 = 1.0 - fx
    gy0 = 1.0 - fy
    w00[...] = jnp.where(v00, at * gx0 * gy0, 0.0)
    w10[...] = jnp.where(v10, at * fx * gy0, 0.0)
    w01[...] = jnp.where(v01, at * gx0 * fy, 0.0)
    w11[...] = jnp.where(v11, at * fx * fy, 0.0)


def _sample(q2, wsxt, bsx, wsyt, bsy, wat, ba2, g, rfx, rfy,
            wf, hf, wi, hi, st, mv, nb):
    r = q2.shape[0]
    br = r // 2
    c = q2.shape[1]
    s128 = 128

    def rowspec(w):
        return pl.BlockSpec((br, w), lambda i: (i, 0))

    def full(a, b):
        return pl.BlockSpec((a, b), lambda i: (0, 0))

    io = jax.ShapeDtypeStruct((r, s128), jnp.int32)
    wo = jax.ShapeDtypeStruct((r, s128), jnp.float32)
    return pl.pallas_call(
        _sample_body,
        grid=(2,),
        in_specs=[
            rowspec(c), full(c, s128), full(1, s128), full(c, s128),
            full(1, s128), full(c, s128), full(1, s128), full(s128, s128),
            rowspec(s128), rowspec(s128), full(1, s128), full(1, s128),
            full(1, s128), full(1, s128), full(1, s128), full(1, s128),
            rowspec(s128),
        ],
        out_specs=[rowspec(s128)] * 8,
        out_shape=[io, io, io, io, wo, wo, wo, wo],
    )(q2, wsxt, bsx, wsyt, bsy, wat, ba2, g, rfx, rfy,
      wf, hf, wi, hi, st, mv, nb)


def _sc_gather(table, idxs, wgts, rq):
    ch = 2
    perq = rq // _NW
    nit = perq // (2 * ch)
    rpc = ch * 128 * 4
    rows = rq * _M

    mesh = plsc.VectorSubcoreMesh(core_axis_name="c", subcore_axis_name="s")

    @functools.partial(
        pl.kernel,
        out_type=jax.ShapeDtypeStruct((rows, _D), jnp.float32),
        mesh=mesh,
        compiler_params=pltpu.CompilerParams(use_tc_tiling_on_sc=False,
                                             needs_layout_passes=False),
        scratch_types=[
            pltpu.VMEM((2, 8, 128), jnp.int32),
            pltpu.VMEM((2, rpc, _D), jnp.bfloat16),
            pltpu.VMEM((2, 8, 128), jnp.float32),
            pltpu.VMEM((ch * _M, _D), jnp.float32),
            pltpu.SemaphoreType.DMA,
            pltpu.SemaphoreType.DMA,
            pltpu.SemaphoreType.DMA,
            pltpu.SemaphoreType.DMA,
        ],
    )
    def k(tab, i0h, i1h, i2h, i3h, w0h, w1h, w2h, w3h, outh,
          idx_v, rows_v, w_v, out_v, sem0, sem1, fsem0, fsem1):
        wid = lax.axis_index("s") * 2 + lax.axis_index("c")
        qbase = wid * perq
        sems = (sem0, sem1)
        fsems = (fsem0, fsem1)
        ihs = (i0h, i1h, i2h, i3h)
        whs = (w0h, w1h, w2h, w3h)

        def fetch(t, b):
            q0 = pl.multiple_of(qbase + t * ch, ch)
            for c in range(4):
                pltpu.async_copy(ihs[c].at[pl.ds(q0, ch)],
                                 idx_v.at[b].at[pl.ds(c * ch, ch)], fsems[b])
                pltpu.async_copy(whs[c].at[pl.ds(q0, ch)],
                                 w_v.at[b].at[pl.ds(c * ch, ch)], fsems[b])

        def fire(b):
            for c in range(4):
                pltpu.make_async_copy(ihs[c].at[pl.ds(0, ch)],
                                      idx_v.at[b].at[pl.ds(c * ch, ch)],
                                      fsems[b]).wait()
                pltpu.make_async_copy(whs[c].at[pl.ds(0, ch)],
                                      w_v.at[b].at[pl.ds(c * ch, ch)],
                                      fsems[b]).wait()
            for sl in range(4 * ch):
                pltpu.async_copy(tab.at[idx_v.at[b].at[sl]],
                                 rows_v.at[b].at[pl.ds(sl * 128, 128)],
                                 sems[b])

        def drain(b):
            pltpu.make_async_copy(tab.at[pl.ds(0, rpc)], rows_v.at[b],
                                  sems[b]).wait()

        def compute_store(t, b):
            rv = rows_v.at[b]
            wv = w_v.at[b]

            def rbody(rr, cr):
                qq = rr // _M
                m16 = (rr - qq * _M) * 16
                a0 = jnp.zeros((16,), jnp.float32)
                a1 = jnp.zeros((16,), jnp.float32)
                for c in range(4):
                    s = c * ch + qq
                    wvec = wv[s, pl.ds(m16, 16)]
                    k0 = s * 128 + m16
                    for jj in range(16):
                        ev, od = plsc.unpack(
                            rv[k0 + jj, :],
                            format=plsc.PackFormat.INTERLEAVED)
                        wsc = wvec[jj]
                        a0 = a0 + wsc * ev
                        a1 = a1 + wsc * od
                out_v[rr, 0:16] = a0
                out_v[rr, 16:32] = a1
                return cr

            lax.fori_loop(0, ch * _M, rbody, 0)
            q0 = pl.multiple_of(qbase + t * ch, ch)
            pltpu.sync_copy(out_v, outh.at[pl.ds(q0 * _M, ch * _M)])

        fetch(0, 0)
        fire(0)

        def it(i, carry):
            t0 = i * 2
            fetch(t0 + 1, 1)
            drain(0)
            fire(1)
            compute_store(t0, 0)

            @pl.when(i < nit - 1)
            def _():
                fetch(t0 + 2, 0)

            drain(1)

            @pl.when(i < nit - 1)
            def _():
                fire(0)

            compute_store(t0 + 1, 1)
            return carry

        lax.fori_loop(0, nit, it, 0)

    return k(table, *idxs, *wgts)


def kernel(query, reference_points, input_flatten, input_space_shape,
           input_level_start_idx, Wv, bv, Ws, bs, Wa, ba, Wo, bo):
    n, lq, c = query.shape
    len_in = input_flatten.shape[1]
    r = n * lq
    rows = r * _M

    q2 = query.reshape(r, c)
    x2 = input_flatten.reshape(n * len_in, c)

    val = _mmb(x2, Wv.T, bv.reshape(1, c), br=512, out_dtype=jnp.bfloat16)
    table = val.reshape(n * len_in * _M, _D)

    wst = Ws.T
    wsxt = wst[:, 0::2]
    wsyt = wst[:, 1::2]
    bsx = bs[0::2].reshape(1, 128)
    bsy = bs[1::2].reshape(1, 128)
    wat = Wa.T
    ba2 = ba.reshape(1, 128)
    lane = jnp.arange(128, dtype=jnp.int32)
    g = (lane[:, None] // 16 == lane[None, :] // 16).astype(jnp.float32)

    rp = reference_points.reshape(r, _L, 2)
    rfx = jnp.tile(jnp.repeat(rp[:, :, 0], _P, axis=1), (1, _M))
    rfy = jnp.tile(jnp.repeat(rp[:, :, 1], _P, axis=1), (1, _M))

    def t128(v):
        return jnp.tile(jnp.repeat(v, _P), (_M,)).reshape(1, 128)

    wf = t128(input_space_shape[:, 1].astype(jnp.float32))
    hf = t128(input_space_shape[:, 0].astype(jnp.float32))
    wi = t128(input_space_shape[:, 1])
    hi = t128(input_space_shape[:, 0])
    st = t128(input_level_start_idx)
    mv = jnp.repeat(jnp.arange(_M, dtype=jnp.int32), _L * _P).reshape(1, 128)
    nb = jnp.broadcast_to(
        ((jnp.arange(r, dtype=jnp.int32) // lq) * len_in)[:, None], (r, 128))

    i00, i10, i01, i11, w00, w10, w01, w11 = _sample(
        q2, wsxt, bsx, wsyt, bsy, wat, ba2, g, rfx, rfy,
        wf, hf, wi, hi, st, mv, nb)

    rq = -(-r // (_NW * 4)) * (_NW * 4)
    padq = rq - r

    def padz(a, dt):
        return jnp.concatenate([a, jnp.zeros((padq, 128), dt)], axis=0)

    idxs = [padz(a, jnp.int32) for a in (i00, i10, i01, i11)]
    wgts = [padz(a, jnp.float32) for a in (w00, w10, w01, w11)]

    out_sc = _sc_gather(table, idxs, wgts, rq)[:rows]

    half = _D // 2
    pcol = jnp.arange(_D)
    chan = jnp.where(pcol < half, 2 * pcol, 2 * (pcol - half) + 1)
    perm = (jnp.arange(c) // _D) * _D + chan[jnp.arange(c) % _D]
    wot_perm = Wo.T[perm, :]
    out = _mmb(out_sc.reshape(r, c), wot_perm, bo.reshape(1, c), br=720)
    return out.reshape(n, lq, c)


# speedup vs baseline: 10.7740x; 1.1326x over previous
import functools

import jax
import jax.numpy as jnp
from jax import lax
from jax.experimental import pallas as pl
from jax.experimental.pallas import tpu as pltpu
from jax.experimental.pallas import tpu_sc as plsc

_M = 8
_L = 4
_P = 4
_D = 32
_LP4 = _L * _P * 4
_NW = 32


def _mmb(x, wt, b, br, out_dtype=jnp.float32):
    r, k = x.shape
    nn = wt.shape[1]

    def body(xr, wr, brf, o):
        o[...] = (jnp.dot(xr[...], wr[...],
                          preferred_element_type=jnp.float32)
                  + brf[...]).astype(out_dtype)

    return pl.pallas_call(
        body,
        grid=(r // br,),
        in_specs=[
            pl.BlockSpec((br, k), lambda i: (i, 0)),
            pl.BlockSpec((k, nn), lambda i: (0, 0)),
            pl.BlockSpec((1, nn), lambda i: (0, 0)),
        ],
        out_specs=pl.BlockSpec((br, nn), lambda i: (i, 0)),
        out_shape=jax.ShapeDtypeStruct((r, nn), out_dtype),
    )(x, wt, b)


def _sample_body(q, wsx, bsx, wsy, bsy, wa, bav, g, rfx, rfy,
                 wf, hf, wi, hi, st, mv, nb,
                 ig0, ig1, wa0, wb0, wa1, wb1):
    qq = q[...]
    ox = jnp.dot(qq, wsx[...], preferred_element_type=jnp.float32) + bsx[...]
    oy = jnp.dot(qq, wsy[...], preferred_element_type=jnp.float32) + bsy[...]
    a = jnp.dot(qq, wa[...], preferred_element_type=jnp.float32) + bav[...]
    e = jnp.exp(a - jnp.max(a, axis=-1, keepdims=True))
    s = jnp.dot(e, g[...], preferred_element_type=jnp.float32)
    at = e / s

    x = rfx[...] * wf[...] + ox - 0.5
    y = rfy[...] * hf[...] + oy - 0.5
    x0f = jnp.floor(x)
    y0f = jnp.floor(y)
    fx = x - x0f
    fy = y - y0f
    x0 = x0f.astype(jnp.int32)
    y0 = y0f.astype(jnp.int32)
    y1 = y0 + 1
    wim1 = wi[...] - 1
    him1 = hi[...] - 1
    vy0 = (y0 >= 0) & (y0 <= him1)
    vy1 = (y1 >= 0) & (y1 <= him1)
    cy0 = jnp.clip(y0, 0, him1)
    cy1 = jnp.clip(y1, 0, him1)
    xs = jnp.clip(x0, 0, wim1 - 1)
    dx = x - xs.astype(jnp.float32)
    wxa = jnp.maximum(0.0, 1.0 - jnp.abs(dx))
    wxb = jnp.maximum(0.0, 1.0 - jnp.abs(dx - 1.0))

    base = nb[...]
    stt = st[...]
    mvv = mv[...]
    wiv = wi[...]

    def mkidx(cy):
        pos = stt + cy * wiv + xs
        return (base + pos) * _M + mvv

    ig0[...] = mkidx(cy0)
    ig1[...] = mkidx(cy1)
    gy0 = jnp.where(vy0, at * (1.0 - fy), 0.0)
    gy1 = jnp.where(vy1, at * fy, 0.0)
    wa0[...] = gy0 * wxa
    wb0[...] = gy0 * wxb
    wa1[...] = gy1 * wxa
    wb1[...] = gy1 * wxb


def _sample(q2, wsxt, bsx, wsyt, bsy, wat, ba2, g, rfx, rfy,
            wf, hf, wi, hi, st, mv, nb):
    r = q2.shape[0]
    br = r // 2
    c = q2.shape[1]
    s128 = 128

    def rowspec(w):
        return pl.BlockSpec((br, w), lambda i: (i, 0))

    def full(a, b):
        return pl.BlockSpec((a, b), lambda i: (0, 0))

    io = jax.ShapeDtypeStruct((r, s128), jnp.int32)
    wo = jax.ShapeDtypeStruct((r, s128), jnp.float32)
    return pl.pallas_call(
        _sample_body,
        grid=(2,),
        in_specs=[
            rowspec(c), full(c, s128), full(1, s128), full(c, s128),
            full(1, s128), full(c, s128), full(1, s128), full(s128, s128),
            rowspec(s128), rowspec(s128), full(1, s128), full(1, s128),
            full(1, s128), full(1, s128), full(1, s128), full(1, s128),
            rowspec(s128),
        ],
        out_specs=[rowspec(s128)] * 6,
        out_shape=[io, io, wo, wo, wo, wo],
    )(q2, wsxt, bsx, wsyt, bsy, wat, ba2, g, rfx, rfy,
      wf, hf, wi, hi, st, mv, nb)


def _sc_gather(table, idxs, wgts, rq):
    ch = 2
    perq = rq // _NW
    nit = perq // (2 * ch)
    rpc = ch * 128 * 2
    rows = rq * _M

    mesh = plsc.VectorSubcoreMesh(core_axis_name="c", subcore_axis_name="s")

    @functools.partial(
        pl.kernel,
        out_type=jax.ShapeDtypeStruct((rows, _D), jnp.float32),
        mesh=mesh,
        compiler_params=pltpu.CompilerParams(use_tc_tiling_on_sc=False,
                                             needs_layout_passes=False),
        scratch_types=[
            pltpu.VMEM((2, 4, 128), jnp.int32),
            pltpu.VMEM((2, rpc, 2 * _D), jnp.bfloat16),
            pltpu.VMEM((2, 8, 128), jnp.float32),
            pltpu.VMEM((ch * _M, _D), jnp.float32),
            pltpu.SemaphoreType.DMA,
            pltpu.SemaphoreType.DMA,
            pltpu.SemaphoreType.DMA,
            pltpu.SemaphoreType.DMA,
        ],
    )
    def k(tab, i0h, i1h, w0h, w1h, w2h, w3h, outh,
          idx_v, rows_v, w_v, out_v, sem0, sem1, fsem0, fsem1):
        wid = lax.axis_index("s") * 2 + lax.axis_index("c")
        qbase = wid * perq
        sems = (sem0, sem1)
        fsems = (fsem0, fsem1)
        ihs = (i0h, i1h)
        whs = (w0h, w1h, w2h, w3h)

        def fetch(t, b):
            q0 = pl.multiple_of(qbase + t * ch, ch)
            for c in range(2):
                pltpu.async_copy(ihs[c].at[pl.ds(q0, ch)],
                                 idx_v.at[b].at[pl.ds(c * ch, ch)], fsems[b])
            for c in range(4):
                pltpu.async_copy(whs[c].at[pl.ds(q0, ch)],
                                 w_v.at[b].at[pl.ds(c * ch, ch)], fsems[b])

        def fire(b):
            for c in range(2):
                pltpu.make_async_copy(ihs[c].at[pl.ds(0, ch)],
                                      idx_v.at[b].at[pl.ds(c * ch, ch)],
                                      fsems[b]).wait()
            for c in range(4):
                pltpu.make_async_copy(whs[c].at[pl.ds(0, ch)],
                                      w_v.at[b].at[pl.ds(c * ch, ch)],
                                      fsems[b]).wait()
            for sl in range(2 * ch):
                pltpu.async_copy(tab.at[idx_v.at[b].at[sl]],
                                 rows_v.at[b].at[pl.ds(sl * 128, 128)],
                                 sems[b])

        def drain(b):
            pltpu.make_async_copy(tab.at[pl.ds(0, rpc)], rows_v.at[b],
                                  sems[b]).wait()

        def compute_store(t, b):
            rv = rows_v.at[b]
            wv = w_v.at[b]

            def rbody(rr, cr):
                qq = rr // _M
                m16 = (rr - qq * _M) * 16
                a0 = jnp.zeros((16,), jnp.float32)
                a1 = jnp.zeros((16,), jnp.float32)
                for yc in range(2):
                    s = yc * ch + qq
                    wveca = wv[2 * yc * ch + qq, pl.ds(m16, 16)]
                    wvecb = wv[(2 * yc + 1) * ch + qq, pl.ds(m16, 16)]
                    k0 = s * 128 + m16
                    for jj in range(16):
                        eva, oda = plsc.unpack(
                            rv[k0 + jj, pl.ds(0, _D)],
                            format=plsc.PackFormat.INTERLEAVED)
                        evb, odb = plsc.unpack(
                            rv[k0 + jj, pl.ds(_D, _D)],
                            format=plsc.PackFormat.INTERLEAVED)
                        wsa = wveca[jj]
                        wsb = wvecb[jj]
                        a0 = a0 + wsa * eva + wsb * evb
                        a1 = a1 + wsa * oda + wsb * odb
                out_v[rr, 0:16] = a0
                out_v[rr, 16:32] = a1
                return cr

            lax.fori_loop(0, ch * _M, rbody, 0)
            q0 = pl.multiple_of(qbase + t * ch, ch)
            pltpu.sync_copy(out_v, outh.at[pl.ds(q0 * _M, ch * _M)])

        fetch(0, 0)
        fire(0)

        def it(i, carry):
            t0 = i * 2
            fetch(t0 + 1, 1)
            drain(0)
            fire(1)
            compute_store(t0, 0)

            @pl.when(i < nit - 1)
            def _():
                fetch(t0 + 2, 0)

            drain(1)

            @pl.when(i < nit - 1)
            def _():
                fire(0)

            compute_store(t0 + 1, 1)
            return carry

        lax.fori_loop(0, nit, it, 0)

    return k(table, *idxs, *wgts)


def kernel(query, reference_points, input_flatten, input_space_shape,
           input_level_start_idx, Wv, bv, Ws, bs, Wa, ba, Wo, bo):
    n, lq, c = query.shape
    len_in = input_flatten.shape[1]
    r = n * lq
    rows = r * _M

    q2 = query.reshape(r, c)
    x2 = input_flatten.reshape(n * len_in, c)

    val = _mmb(x2, Wv.T, bv.reshape(1, c), br=512, out_dtype=jnp.bfloat16)
    table = val.reshape(n * len_in * _M, _D)

    wst = Ws.T
    wsxt = wst[:, 0::2]
    wsyt = wst[:, 1::2]
    bsx = bs[0::2].reshape(1, 128)
    bsy = bs[1::2].reshape(1, 128)
    wat = Wa.T
    ba2 = ba.reshape(1, 128)
    lane = jnp.arange(128, dtype=jnp.int32)
    g = (lane[:, None] // 16 == lane[None, :] // 16).astype(jnp.float32)

    rp = reference_points.reshape(r, _L, 2)
    rfx = jnp.tile(jnp.repeat(rp[:, :, 0], _P, axis=1), (1, _M))
    rfy = jnp.tile(jnp.repeat(rp[:, :, 1], _P, axis=1), (1, _M))

    def t128(v):
        return jnp.tile(jnp.repeat(v, _P), (_M,)).reshape(1, 128)

    wf = t128(input_space_shape[:, 1].astype(jnp.float32))
    hf = t128(input_space_shape[:, 0].astype(jnp.float32))
    wi = t128(input_space_shape[:, 1])
    hi = t128(input_space_shape[:, 0])
    st = t128(input_level_start_idx)
    mv = jnp.repeat(jnp.arange(_M, dtype=jnp.int32), _L * _P).reshape(1, 128)
    nb = jnp.broadcast_to(
        ((jnp.arange(r, dtype=jnp.int32) // lq) * len_in)[:, None], (r, 128))

    ig0, ig1, wa0, wb0, wa1, wb1 = _sample(
        q2, wsxt, bsx, wsyt, bsy, wat, ba2, g, rfx, rfy,
        wf, hf, wi, hi, st, mv, nb)

    tab2 = jnp.concatenate(
        [table, jnp.concatenate([table[_M:], table[:_M]], axis=0)], axis=1)

    rq = -(-r // (_NW * 4)) * (_NW * 4)
    padq = rq - r

    def padz(a, dt):
        return jnp.concatenate([a, jnp.zeros((padq, 128), dt)], axis=0)

    idxs = [padz(a, jnp.int32) for a in (ig0, ig1)]
    wgts = [padz(a, jnp.float32) for a in (wa0, wb0, wa1, wb1)]

    out_sc = _sc_gather(tab2, idxs, wgts, rq)[:rows]

    half = _D // 2
    pcol = jnp.arange(_D)
    chan = jnp.where(pcol < half, 2 * pcol, 2 * (pcol - half) + 1)
    perm = (jnp.arange(c) // _D) * _D + chan[jnp.arange(c) % _D]
    wot_perm = Wo.T[perm, :]
    out = _mmb(out_sc.reshape(r, c), wot_perm, bo.reshape(1, c), br=720)
    return out.reshape(n, lq, c)
